# Initial kernel scaffold; baseline (speedup 1.0000x reference)
import jax
import jax.numpy as jnp
from jax.experimental import pallas as pl

G, V = 8, 512
GV = G * V
D = 128
TT = 512


def _vq_kernel(x_ref, w_ref, b_ref, cb_ref, logits_ref, out_ref):
    x = x_ref[0]
    logits = jax.lax.dot_general(
        w_ref[...], x, (((1,), (1,)), ((), ())),
        preferred_element_type=jnp.float32)
    logits = logits + b_ref[...]
    logits_ref[0] = logits
    for g in range(G):
        lg = logits[g * V:(g + 1) * V, :]
        idx = jnp.argmax(lg, axis=0)
        onehot = (jax.lax.broadcasted_iota(jnp.int32, (TT, V), 1)
                  == idx[:, None]).astype(jnp.float32)
        cb_g = cb_ref[g * D:(g + 1) * D, :]
        hard = jax.lax.dot_general(
            onehot, cb_g, (((1,), (1,)), ((), ())),
            preferred_element_type=jnp.float32)
        out_ref[0, :, g * D:(g + 1) * D] = hard


def kernel(inputs, W, b, codebooks, temp):
    del temp
    bsize, timesteps, channels = inputs.shape
    b2 = b.reshape(GV, 1)
    cb = codebooks.reshape(G * D, V)
    logits_flat, out = pl.pallas_call(
        _vq_kernel,
        grid=(bsize, timesteps // TT),
        in_specs=[
            pl.BlockSpec((1, TT, channels), lambda i, j: (i, j, 0)),
            pl.BlockSpec((GV, channels), lambda i, j: (0, 0)),
            pl.BlockSpec((GV, 1), lambda i, j: (0, 0)),
            pl.BlockSpec((G * D, V), lambda i, j: (0, 0)),
        ],
        out_specs=[
            pl.BlockSpec((1, GV, TT), lambda i, j: (i, 0, j)),
            pl.BlockSpec((1, TT, channels), lambda i, j: (i, j, 0)),
        ],
        out_shape=[
            jax.ShapeDtypeStruct((bsize, GV, timesteps), jnp.float32),
            jax.ShapeDtypeStruct((bsize, timesteps, channels), jnp.float32),
        ],
    )(inputs, W, b2, cb)
    logits = logits_flat.reshape(bsize, G, V, timesteps)
    return out, logits


# speedup vs baseline: 4.0159x; 4.0159x over previous
import jax
import jax.numpy as jnp
from jax.experimental import pallas as pl

G, V = 8, 512
GV = G * V
D = 128
TT = 256


def _vq_kernel(x_ref, w_ref, b_ref, cb_ref, logits_ref, out_ref):
    x = x_ref[0]
    logits = jax.lax.dot_general(
        w_ref[...], x, (((1,), (1,)), ((), ())),
        preferred_element_type=jnp.float32)
    logits = logits + b_ref[...]
    logits_ref[0] = logits
    for g in range(G):
        lg = logits[g * V:(g + 1) * V, :]
        idx = jnp.argmax(lg, axis=0)
        onehot = (jax.lax.broadcasted_iota(jnp.int32, (TT, V), 1)
                  == idx[:, None]).astype(jnp.float32)
        cb_g = cb_ref[g * D:(g + 1) * D, :]
        hard = jax.lax.dot_general(
            onehot, cb_g, (((1,), (1,)), ((), ())),
            preferred_element_type=jnp.float32)
        out_ref[0, :, g * D:(g + 1) * D] = hard


def kernel(inputs, W, b, codebooks, temp):
    del temp
    bsize, timesteps, channels = inputs.shape
    b2 = b.reshape(GV, 1)
    cb = codebooks.reshape(G * D, V)
    logits_flat, out = pl.pallas_call(
        _vq_kernel,
        grid=(bsize, timesteps // TT),
        in_specs=[
            pl.BlockSpec((1, TT, channels), lambda i, j: (i, j, 0)),
            pl.BlockSpec((GV, channels), lambda i, j: (0, 0)),
            pl.BlockSpec((GV, 1), lambda i, j: (0, 0)),
            pl.BlockSpec((G * D, V), lambda i, j: (0, 0)),
        ],
        out_specs=[
            pl.BlockSpec((1, GV, TT), lambda i, j: (i, 0, j)),
            pl.BlockSpec((1, TT, channels), lambda i, j: (i, j, 0)),
        ],
        out_shape=[
            jax.ShapeDtypeStruct((bsize, GV, timesteps), jnp.float32),
            jax.ShapeDtypeStruct((bsize, timesteps, channels), jnp.float32),
        ],
    )(inputs, W, b2, cb)
    logits = logits_flat.reshape(bsize, G, V, timesteps)
    return out, logits
